# Initial kernel scaffold; baseline (speedup 1.0000x reference)
import functools

import jax
import jax.numpy as jnp
from jax import lax
from jax.experimental import pallas as pl
from jax.experimental.pallas import tpu as pltpu
from jax.experimental.pallas import tpu_sc as plsc

_N = 10000
_D = 128
_E = 320000
_NC = 2
_NS = 16
_NW = _NC * _NS
_EW = _E // _NW
_CH = 80
_NCH = _EW // _CH
_DEG_PAD = 10240
_DEG_W = _DEG_PAD // _NS
_ROWS_W = _N // _NS

_MESH = plsc.VectorSubcoreMesh(core_axis_name="c", subcore_axis_name="s")


@functools.partial(
    pl.kernel,
    out_type=jax.ShapeDtypeStruct((_NC, _DEG_PAD), jnp.float32),
    mesh=_MESH,
    scratch_types=[
        pltpu.VMEM((_NCH, _CH), jnp.int32),
        pltpu.VMEM((_CH,), jnp.float32),
        pltpu.VMEM((_DEG_W,), jnp.float32),
        pltpu.VMEM_SHARED((_DEG_PAD,), jnp.float32),
        pltpu.SemaphoreType.DMA,
    ],
)
def _sc_degree(dst_hbm, deg_hbm, idx_v, ones_v, z_v, acc_sh, sem):
    cid = lax.axis_index("c")
    sid = lax.axis_index("s")
    wid = cid * _NS + sid

    for t in range(_CH // 16):
        ones_v[pl.ds(t * 16, 16)] = jnp.ones((16,), jnp.float32)

    def zfill(t, carry):
        z_v[pl.ds(t * 16, 16)] = jnp.zeros((16,), jnp.float32)
        return carry

    lax.fori_loop(0, _DEG_W // 16, zfill, 0)
    pltpu.sync_copy(z_v, acc_sh.at[pl.ds(sid * _DEG_W, _DEG_W)])
    plsc.subcore_barrier()

    pltpu.sync_copy(dst_hbm.at[wid], idx_v)

    def body(j, carry):
        pltpu.sync_copy(ones_v, acc_sh.at[idx_v.at[j]], add=True)
        return carry

    lax.fori_loop(0, _NCH, body, 0)
    plsc.subcore_barrier()
    pltpu.sync_copy(acc_sh.at[pl.ds(sid * _DEG_W, _DEG_W)],
                    deg_hbm.at[cid, pl.ds(sid * _DEG_W, _DEG_W)])


@functools.partial(
    pl.kernel,
    out_type=jax.ShapeDtypeStruct((_NC, _N, _D), jnp.float32),
    mesh=_MESH,
    scratch_types=[
        pltpu.VMEM((_NCH, _CH), jnp.int32),
        pltpu.VMEM((_NCH, _CH), jnp.int32),
        pltpu.VMEM((_CH, _D), jnp.float32),
        pltpu.VMEM((_NCH, _D), jnp.float32),
        pltpu.VMEM_SHARED((_N, _D), jnp.float32),
        pltpu.SemaphoreType.DMA,
    ],
)
def _sc_scatter(g_hbm, src_hbm, dst_hbm, out_hbm,
                sidx_v, didx_v, rows_v, zrow_v, acc_sh, sem):
    cid = lax.axis_index("c")
    sid = lax.axis_index("s")
    wid = cid * _NS + sid

    def zfill(r, carry):
        for t in range(_D // 16):
            zrow_v[r, pl.ds(t * 16, 16)] = jnp.zeros((16,), jnp.float32)
        return carry

    lax.fori_loop(0, _NCH, zfill, 0)
    for t in range(_ROWS_W // _NCH):
        pltpu.sync_copy(zrow_v,
                        acc_sh.at[pl.ds(sid * _ROWS_W + t * _NCH, _NCH)])
    plsc.subcore_barrier()

    pltpu.sync_copy(src_hbm.at[wid], sidx_v)
    pltpu.sync_copy(dst_hbm.at[wid], didx_v)

    def body(j, carry):
        pltpu.async_copy(g_hbm.at[sidx_v.at[j]], rows_v, sem).wait()
        pltpu.sync_copy(rows_v, acc_sh.at[didx_v.at[j]], add=True)
        return carry

    lax.fori_loop(0, _NCH, body, 0)
    plsc.subcore_barrier()
    pltpu.sync_copy(acc_sh.at[pl.ds(sid * _ROWS_W, _ROWS_W)],
                    out_hbm.at[cid, pl.ds(sid * _ROWS_W, _ROWS_W)])


def _tc1_body(x_ref, w_ref, d0_ref, d1_ref, g_ref, dinv_ref):
    deg = d0_ref[...] + d1_ref[...] + 1.0
    dinv = lax.rsqrt(deg)
    h = jnp.dot(x_ref[...], w_ref[...], preferred_element_type=jnp.float32)
    g_ref[...] = h * dinv
    dinv_ref[...] = dinv


def _tc2_body(s0_ref, s1_ref, g_ref, dinv_ref, b_ref, w_ref, gn_ref):
    dinv = dinv_ref[...]
    agg = dinv * (s0_ref[...] + s1_ref[...] + g_ref[...]) + b_ref[...]
    h = jnp.maximum(agg, 0.0)
    hw = jnp.dot(h, w_ref[...], preferred_element_type=jnp.float32)
    gn_ref[...] = hw * dinv


def _tc3_body(s0_ref, s1_ref, g_ref, dinv_ref, b_ref, proto_ref, wlt_ref,
              logits_ref, probs_ref, emb_ref, dist_ref):
    agg = dinv_ref[...] * (s0_ref[...] + s1_ref[...] + g_ref[...]) + b_ref[...]
    emb = jnp.maximum(agg, 0.0)
    proto = proto_ref[...]
    xp = lax.dot_general(emb, proto, (((1,), (1,)), ((), ())),
                         preferred_element_type=jnp.float32)
    psq = jnp.sum(proto * proto, axis=1)[None, :]
    esq = jnp.sum(emb * emb, axis=1, keepdims=True)
    dist = -2.0 * xp + esq + psq
    sim = jnp.log((dist + 1.0) / (dist + 1e-4))
    logits = jnp.dot(sim, wlt_ref[...], preferred_element_type=jnp.float32)
    col = lax.broadcasted_iota(jnp.int32, logits.shape, 1)
    ml = jnp.where(col < 10, logits, -1e30)
    m = jnp.max(ml, axis=1, keepdims=True)
    e = jnp.exp(ml - m)
    p = e / jnp.sum(e, axis=1, keepdims=True)
    logits_ref[...] = logits[:, :10]
    probs_ref[...] = p[:, :10]
    emb_ref[...] = emb
    dist_ref[...] = dist[:, :50]


_BLK = 1000
_GRID = _N // _BLK

_row_spec = pl.BlockSpec((_BLK, _D), lambda i: (i, 0))
_col_spec = pl.BlockSpec((_BLK, 1), lambda i: (i, 0))
_w_spec = pl.BlockSpec((_D, _D), lambda i: (0, 0))
_b_spec = pl.BlockSpec((1, _D), lambda i: (0, 0))

_tc1 = pl.pallas_call(
    _tc1_body,
    grid=(_GRID,),
    in_specs=[_row_spec, _w_spec, _col_spec, _col_spec],
    out_specs=[_row_spec, _col_spec],
    out_shape=[jax.ShapeDtypeStruct((_N, _D), jnp.float32),
               jax.ShapeDtypeStruct((_N, 1), jnp.float32)],
)

_tc2 = pl.pallas_call(
    _tc2_body,
    grid=(_GRID,),
    in_specs=[_row_spec, _row_spec, _row_spec, _col_spec, _b_spec, _w_spec],
    out_specs=[_row_spec],
    out_shape=[jax.ShapeDtypeStruct((_N, _D), jnp.float32)],
)

_tc3 = pl.pallas_call(
    _tc3_body,
    grid=(_GRID,),
    in_specs=[_row_spec, _row_spec, _row_spec, _col_spec, _b_spec,
              pl.BlockSpec((64, _D), lambda i: (0, 0)),
              pl.BlockSpec((64, 16), lambda i: (0, 0))],
    out_specs=[pl.BlockSpec((_BLK, 10), lambda i: (i, 0)),
               pl.BlockSpec((_BLK, 10), lambda i: (i, 0)),
               _row_spec,
               pl.BlockSpec((_BLK, 50), lambda i: (i, 0))],
    out_shape=[jax.ShapeDtypeStruct((_N, 10), jnp.float32),
               jax.ShapeDtypeStruct((_N, 10), jnp.float32),
               jax.ShapeDtypeStruct((_N, _D), jnp.float32),
               jax.ShapeDtypeStruct((_N, 50), jnp.float32)],
)


def kernel(x, edge_index, W1, b1, W2, b2, W3, b3, proto, W_last):
    src = edge_index[0].astype(jnp.int32).reshape(_NW, _NCH, _CH)
    dst = edge_index[1].astype(jnp.int32).reshape(_NW, _NCH, _CH)

    deg2 = _sc_degree(dst)
    deg0 = deg2[0, :_N].reshape(_N, 1)
    deg1 = deg2[1, :_N].reshape(_N, 1)

    b1r = b1.reshape(1, _D)
    b2r = b2.reshape(1, _D)
    b3r = b3.reshape(1, _D)
    proto_pad = jnp.zeros((64, _D), jnp.float32).at[:50].set(proto)
    wlt_pad = jnp.zeros((64, 16), jnp.float32).at[:50, :10].set(W_last.T)

    g1, dinv = _tc1(x, W1, deg0, deg1)

    s = _sc_scatter(g1, src, dst)
    g2, = _tc2(s[0], s[1], g1, dinv, b1r, W2)

    s = _sc_scatter(g2, src, dst)
    g3, = _tc2(s[0], s[1], g2, dinv, b2r, W3)

    s = _sc_scatter(g3, src, dst)
    logits, probs, emb, dist = _tc3(s[0], s[1], g3, dinv, b3r,
                                    proto_pad, wlt_pad)
    return (logits, probs, emb, dist)


# speedup vs baseline: 15.9698x; 15.9698x over previous
import functools

import jax
import jax.numpy as jnp
from jax import lax
from jax.experimental import pallas as pl
from jax.experimental.pallas import tpu as pltpu
from jax.experimental.pallas import tpu_sc as plsc

_N = 10000
_D = 128
_E = 320000
_NC = 2
_NS = 16
_NW = _NC * _NS
_EW = _E // _NW
_CH = 80
_NCH = _EW // _CH
_DEG_PAD = 10240
_DEG_W = _DEG_PAD // _NS
_NPADR = 10240
_ROWS_W = _NPADR // _NS
_ZR = 128

@functools.cache
def _sc_kernels():
    mesh = plsc.VectorSubcoreMesh(core_axis_name="c", subcore_axis_name="s")

    sc_degree = functools.partial(
        pl.kernel,
        out_type=jax.ShapeDtypeStruct((_NC, 1, _DEG_PAD), jnp.float32),
        mesh=mesh,
        scratch_types=[
            pltpu.VMEM((_NCH, _CH), jnp.int32),
            pltpu.VMEM((_CH,), jnp.float32),
            pltpu.VMEM((_DEG_W,), jnp.float32),
            pltpu.VMEM_SHARED((_DEG_PAD,), jnp.float32),
            pltpu.SemaphoreType.DMA,
        ],
    )(_sc_degree_body)

    sc_scatter = functools.partial(
        pl.kernel,
        out_type=jax.ShapeDtypeStruct((_NC, _NPADR, _D), jnp.float32),
        mesh=mesh,
        scratch_types=[
            pltpu.VMEM((_NCH, _CH), jnp.int32),
            pltpu.VMEM((_NCH, _CH), jnp.int32),
            pltpu.VMEM((_CH, _D), jnp.float32),
            pltpu.VMEM_SHARED((_NPADR, _D), jnp.float32),
            pltpu.SemaphoreType.DMA,
        ],
    )(_sc_scatter_body)

    return sc_degree, sc_scatter


def _sc_degree_body(dst_hbm, deg_hbm, idx_v, ones_v, z_v, acc_sh, sem):
    cid = lax.axis_index("c")
    sid = lax.axis_index("s")
    wid = cid * _NS + sid

    for t in range(_CH // 16):
        ones_v[pl.ds(t * 16, 16)] = jnp.ones((16,), jnp.float32)

    def zfill(t, carry):
        z_v[pl.ds(t * 16, 16)] = jnp.zeros((16,), jnp.float32)
        return carry

    lax.fori_loop(0, _DEG_W // 16, zfill, 0)
    pltpu.sync_copy(z_v, acc_sh.at[pl.ds(sid * _DEG_W, _DEG_W)])
    plsc.subcore_barrier()

    pltpu.sync_copy(dst_hbm.at[wid], idx_v)

    def body(j, carry):
        pltpu.sync_copy(ones_v, acc_sh.at[idx_v.at[j]], add=True)
        return carry

    lax.fori_loop(0, _NCH, body, 0)
    plsc.subcore_barrier()
    pltpu.sync_copy(acc_sh.at[pl.ds(sid * _DEG_W, _DEG_W)],
                    deg_hbm.at[cid, 0, pl.ds(sid * _DEG_W, _DEG_W)])


def _sc_scatter_body(g_hbm, src_hbm, dst_hbm, out_hbm,
                     sidx_v, didx_v, rows_v, acc_sh, sem):
    cid = lax.axis_index("c")
    sid = lax.axis_index("s")
    wid = cid * _NS + sid

    def zfill(r, carry):
        for t in range(_D // 16):
            rows_v[r, pl.ds(t * 16, 16)] = jnp.zeros((16,), jnp.float32)
        return carry

    lax.fori_loop(0, _CH, zfill, 0)
    for t in range(_ROWS_W // _CH):
        pltpu.sync_copy(rows_v,
                        acc_sh.at[pl.ds(sid * _ROWS_W + t * _CH, _CH)])
    plsc.subcore_barrier()

    pltpu.sync_copy(src_hbm.at[wid], sidx_v)
    pltpu.sync_copy(dst_hbm.at[wid], didx_v)

    def body(j, carry):
        pltpu.async_copy(g_hbm.at[sidx_v.at[j]], rows_v, sem).wait()
        pltpu.sync_copy(rows_v, acc_sh.at[didx_v.at[j]], add=True)
        return carry

    lax.fori_loop(0, _NCH, body, 0)
    plsc.subcore_barrier()
    pltpu.sync_copy(acc_sh.at[pl.ds(sid * _ROWS_W, _ROWS_W)],
                    out_hbm.at[cid, pl.ds(sid * _ROWS_W, _ROWS_W)])


def _tc1_body(x_ref, w_ref, d0_ref, d1_ref, g_ref, dinv_ref):
    deg = d0_ref[...] + d1_ref[...] + 1.0
    dinv = lax.rsqrt(deg)
    h = jnp.dot(x_ref[...], w_ref[...], preferred_element_type=jnp.float32)
    g_ref[...] = h * dinv
    dinv_ref[...] = dinv


def _tc2_body(s0_ref, s1_ref, g_ref, dinv_ref, b_ref, w_ref, gn_ref):
    dinv = dinv_ref[...]
    agg = dinv * (s0_ref[...] + s1_ref[...] + g_ref[...]) + b_ref[...]
    h = jnp.maximum(agg, 0.0)
    hw = jnp.dot(h, w_ref[...], preferred_element_type=jnp.float32)
    gn_ref[...] = hw * dinv


def _tc3_body(s0_ref, s1_ref, g_ref, dinv_ref, b_ref, proto_ref, wlt_ref,
              logits_ref, probs_ref, emb_ref, dist_ref):
    agg = dinv_ref[...] * (s0_ref[...] + s1_ref[...] + g_ref[...]) + b_ref[...]
    emb = jnp.maximum(agg, 0.0)
    proto = proto_ref[...]
    xp = lax.dot_general(emb, proto, (((1,), (1,)), ((), ())),
                         preferred_element_type=jnp.float32)
    psq = jnp.sum(proto * proto, axis=1)[None, :]
    esq = jnp.sum(emb * emb, axis=1, keepdims=True)
    dist = -2.0 * xp + esq + psq
    sim = jnp.log((dist + 1.0) / (dist + 1e-4))
    logits = jnp.dot(sim, wlt_ref[...], preferred_element_type=jnp.float32)
    col = lax.broadcasted_iota(jnp.int32, logits.shape, 1)
    ml = jnp.where(col < 10, logits, -1e30)
    m = jnp.max(ml, axis=1, keepdims=True)
    e = jnp.exp(ml - m)
    p = e / jnp.sum(e, axis=1, keepdims=True)
    logits_ref[...] = logits[:, :10]
    probs_ref[...] = p[:, :10]
    emb_ref[...] = emb
    dist_ref[...] = dist[:, :50]


_BLK = 1000
_GRID = _N // _BLK

_row_spec = pl.BlockSpec((_BLK, _D), lambda i: (i, 0))
_col_spec = pl.BlockSpec((_BLK, 1), lambda i: (i, 0))
_w_spec = pl.BlockSpec((_D, _D), lambda i: (0, 0))
_b_spec = pl.BlockSpec((1, _D), lambda i: (0, 0))

_tc1 = pl.pallas_call(
    _tc1_body,
    grid=(_GRID,),
    in_specs=[_row_spec, _w_spec, _col_spec, _col_spec],
    out_specs=[_row_spec, _col_spec],
    out_shape=[jax.ShapeDtypeStruct((_N, _D), jnp.float32),
               jax.ShapeDtypeStruct((_N, 1), jnp.float32)],
)

_tc2 = pl.pallas_call(
    _tc2_body,
    grid=(_GRID,),
    in_specs=[_row_spec, _row_spec, _row_spec, _col_spec, _b_spec, _w_spec],
    out_specs=[_row_spec],
    out_shape=[jax.ShapeDtypeStruct((_N, _D), jnp.float32)],
)

_tc3 = pl.pallas_call(
    _tc3_body,
    grid=(_GRID,),
    in_specs=[_row_spec, _row_spec, _row_spec, _col_spec, _b_spec,
              pl.BlockSpec((64, _D), lambda i: (0, 0)),
              pl.BlockSpec((64, 16), lambda i: (0, 0))],
    out_specs=[pl.BlockSpec((_BLK, 10), lambda i: (i, 0)),
               pl.BlockSpec((_BLK, 10), lambda i: (i, 0)),
               _row_spec,
               pl.BlockSpec((_BLK, 50), lambda i: (i, 0))],
    out_shape=[jax.ShapeDtypeStruct((_N, 10), jnp.float32),
               jax.ShapeDtypeStruct((_N, 10), jnp.float32),
               jax.ShapeDtypeStruct((_N, _D), jnp.float32),
               jax.ShapeDtypeStruct((_N, 50), jnp.float32)],
)


def kernel(x, edge_index, W1, b1, W2, b2, W3, b3, proto, W_last):
    src = edge_index[0].astype(jnp.int32).reshape(_NW, _NCH, _CH)
    dst = edge_index[1].astype(jnp.int32).reshape(_NW, _NCH, _CH)

    _sc_degree, _sc_scatter = _sc_kernels()
    deg2 = _sc_degree(dst)
    deg0 = deg2[0, 0, :_N].reshape(_N, 1)
    deg1 = deg2[1, 0, :_N].reshape(_N, 1)

    b1r = b1.reshape(1, _D)
    b2r = b2.reshape(1, _D)
    b3r = b3.reshape(1, _D)
    proto_pad = jnp.zeros((64, _D), jnp.float32).at[:50].set(proto)
    wlt_pad = jnp.zeros((64, 16), jnp.float32).at[:50, :10].set(W_last.T)

    g1, dinv = _tc1(x, W1, deg0, deg1)

    s = _sc_scatter(g1, src, dst)
    g2, = _tc2(s[0, :_N], s[1, :_N], g1, dinv, b1r, W2)

    s = _sc_scatter(g2, src, dst)
    g3, = _tc2(s[0, :_N], s[1, :_N], g2, dinv, b2r, W3)

    s = _sc_scatter(g3, src, dst)
    logits, probs, emb, dist = _tc3(s[0, :_N], s[1, :_N], g3, dinv, b3r,
                                    proto_pad, wlt_pad)
    return (logits, probs, emb, dist)


# speedup vs baseline: 20.1183x; 1.2598x over previous
import functools

import jax
import jax.numpy as jnp
from jax import lax
from jax.experimental import pallas as pl
from jax.experimental.pallas import tpu as pltpu
from jax.experimental.pallas import tpu_sc as plsc

_N = 10000
_D = 128
_E = 320000
_NC = 2
_NS = 16
_NW = _NC * _NS
_EW = _E // _NW
_CH = 64
_EWP = 10112
_EPAD = _EWP - _EW
_NCH = _EWP // _CH
_DEG_PAD = 10240
_DEG_W = _DEG_PAD // _NS
_NPADR = 10240
_ROWS_W = _NPADR // _NS
_ZR = 128

@functools.cache
def _sc_kernels():
    mesh = plsc.VectorSubcoreMesh(core_axis_name="c", subcore_axis_name="s")

    sc_degree = functools.partial(
        pl.kernel,
        out_type=jax.ShapeDtypeStruct((_NC, 1, _DEG_PAD), jnp.float32),
        mesh=mesh,
        scratch_types=[
            pltpu.VMEM((_NCH, _CH), jnp.int32),
            pltpu.VMEM((_CH,), jnp.float32),
            pltpu.VMEM((_DEG_W,), jnp.float32),
            pltpu.VMEM_SHARED((_DEG_PAD,), jnp.float32),
            pltpu.SemaphoreType.DMA,
        ],
    )(_sc_degree_body)

    sc_scatter = functools.partial(
        pl.kernel,
        out_type=jax.ShapeDtypeStruct((_NC, _NPADR, _D), jnp.float32),
        mesh=mesh,
        scratch_types=[
            pltpu.VMEM((_EWP,), jnp.int32),
            pltpu.VMEM((_NCH, _CH), jnp.int32),
            pltpu.VMEM((_CH, _D), jnp.float32),
            pltpu.VMEM((_CH, _D), jnp.float32),
            pltpu.VMEM_SHARED((_NPADR, _D), jnp.float32),
            pltpu.SemaphoreType.DMA,
            pltpu.SemaphoreType.DMA,
            pltpu.SemaphoreType.DMA,
            pltpu.SemaphoreType.DMA,
            pltpu.SemaphoreType.DMA,
        ],
    )(_sc_scatter_body)

    return sc_degree, sc_scatter


def _sc_degree_body(dst_hbm, deg_hbm, idx_v, ones_v, z_v, acc_sh, sem):
    cid = lax.axis_index("c")
    sid = lax.axis_index("s")
    wid = cid * _NS + sid

    for t in range(0, _CH - 15, 16):
        ones_v[pl.ds(t, 16)] = jnp.ones((16,), jnp.float32)
    if _CH % 16:
        ones_v[pl.ds(_CH - 16, 16)] = jnp.ones((16,), jnp.float32)

    def zfill(t, carry):
        z_v[pl.ds(t * 16, 16)] = jnp.zeros((16,), jnp.float32)
        return carry

    lax.fori_loop(0, _DEG_W // 16, zfill, 0)
    pltpu.sync_copy(z_v, acc_sh.at[pl.ds(sid * _DEG_W, _DEG_W)])
    plsc.subcore_barrier()

    pltpu.sync_copy(dst_hbm.at[wid], idx_v)

    def body(j, carry):
        pltpu.sync_copy(ones_v, acc_sh.at[idx_v.at[j]], add=True)
        return carry

    lax.fori_loop(0, _NCH, body, 0)
    plsc.subcore_barrier()
    pltpu.sync_copy(acc_sh.at[pl.ds(sid * _DEG_W, _DEG_W)],
                    deg_hbm.at[cid, 0, pl.ds(sid * _DEG_W, _DEG_W)])


def _sc_scatter_body(g_hbm, src_hbm, dst_hbm, out_hbm,
                     sidx_v, didx_v, rows_a, rows_b, acc_sh,
                     isem, gsa, gsb, ssa, ssb):
    cid = lax.axis_index("c")
    sid = lax.axis_index("s")
    wid = cid * _NS + sid

    ci_s = pltpu.make_async_copy(src_hbm.at[wid, 0], sidx_v, isem)
    ci_d = pltpu.make_async_copy(dst_hbm.at[wid], didx_v, isem)
    ci_s.start()
    ci_d.start()

    def zfill(r, carry):
        for t in range(_D // 16):
            rows_a[r, pl.ds(t * 16, 16)] = jnp.zeros((16,), jnp.float32)
        return carry

    lax.fori_loop(0, _CH, zfill, 0)
    for t in range(_ROWS_W // _CH):
        pltpu.sync_copy(rows_a,
                        acc_sh.at[pl.ds(sid * _ROWS_W + t * _CH, _CH)])
    ci_s.wait()
    ci_d.wait()
    plsc.subcore_barrier()

    def g_start(j, buf, sem):
        pltpu.async_copy(g_hbm.at[sidx_v.at[pl.ds(j * _CH, _CH)]], buf, sem)

    def g_wait(j, buf, sem):
        pltpu.make_async_copy(g_hbm.at[sidx_v.at[pl.ds(j * _CH, _CH)]],
                              buf, sem).wait()

    def s_start(j, buf, sem):
        pltpu.async_copy(buf, acc_sh.at[didx_v.at[j]], sem, add=True)

    def s_wait(j, buf, sem):
        pltpu.make_async_copy(buf, acc_sh.at[didx_v.at[j]], sem).wait()

    g_start(0, rows_a, gsa)
    g_start(1, rows_b, gsb)

    def body(i, carry):
        j = 2 * i
        g_wait(j, rows_a, gsa)
        s_start(j, rows_a, ssa)
        g_wait(j + 1, rows_b, gsb)
        s_start(j + 1, rows_b, ssb)
        s_wait(j, rows_a, ssa)

        @pl.when(j + 2 < _NCH)
        def _():
            g_start(j + 2, rows_a, gsa)

        s_wait(j + 1, rows_b, ssb)

        @pl.when(j + 3 < _NCH)
        def _():
            g_start(j + 3, rows_b, gsb)

        return carry

    lax.fori_loop(0, _NCH // 2, body, 0)
    plsc.subcore_barrier()
    pltpu.sync_copy(acc_sh.at[pl.ds(sid * _ROWS_W, _ROWS_W)],
                    out_hbm.at[cid, pl.ds(sid * _ROWS_W, _ROWS_W)])


def _tc1_body(x_ref, w_ref, d0_ref, d1_ref, g_ref, dinv_ref):
    deg = d0_ref[...] + d1_ref[...] + 1.0
    dinv = lax.rsqrt(deg)
    h = jnp.dot(x_ref[...], w_ref[...], preferred_element_type=jnp.float32)
    g_ref[...] = h * dinv
    dinv_ref[...] = dinv


def _tc2_body(s_ref, g_ref, dinv_ref, b_ref, w_ref, gn_ref):
    dinv = dinv_ref[...]
    agg = dinv * (s_ref[0] + s_ref[1] + g_ref[...]) + b_ref[...]
    h = jnp.maximum(agg, 0.0)
    hw = jnp.dot(h, w_ref[...], preferred_element_type=jnp.float32)
    gn_ref[...] = hw * dinv


def _tc3_body(s_ref, g_ref, dinv_ref, b_ref, proto_ref, wlt_ref,
              logits_ref, probs_ref, emb_ref, dist_ref):
    agg = dinv_ref[...] * (s_ref[0] + s_ref[1] + g_ref[...]) + b_ref[...]
    emb = jnp.maximum(agg, 0.0)
    proto = proto_ref[...]
    xp = lax.dot_general(emb, proto, (((1,), (1,)), ((), ())),
                         preferred_element_type=jnp.float32)
    psq = jnp.sum(proto * proto, axis=1)[None, :]
    esq = jnp.sum(emb * emb, axis=1, keepdims=True)
    dist = -2.0 * xp + esq + psq
    sim = jnp.log((dist + 1.0) / (dist + 1e-4))
    logits = jnp.dot(sim, wlt_ref[...], preferred_element_type=jnp.float32)
    col = lax.broadcasted_iota(jnp.int32, logits.shape, 1)
    ml = jnp.where(col < 10, logits, -1e30)
    m = jnp.max(ml, axis=1, keepdims=True)
    e = jnp.exp(ml - m)
    p = e / jnp.sum(e, axis=1, keepdims=True)
    logits_ref[...] = logits[:, :10]
    probs_ref[...] = p[:, :10]
    emb_ref[...] = emb
    dist_ref[...] = dist[:, :50]


_BLK = 1000
_GRID = _N // _BLK

_row_spec = pl.BlockSpec((_BLK, _D), lambda i: (i, 0))
_col_spec = pl.BlockSpec((_BLK, 1), lambda i: (i, 0))
_w_spec = pl.BlockSpec((_D, _D), lambda i: (0, 0))
_b_spec = pl.BlockSpec((1, _D), lambda i: (0, 0))
_s_spec = pl.BlockSpec((_NC, _BLK, _D), lambda i: (0, i, 0))

_tc1 = pl.pallas_call(
    _tc1_body,
    grid=(_GRID,),
    in_specs=[_row_spec, _w_spec, _col_spec, _col_spec],
    out_specs=[_row_spec, _col_spec],
    out_shape=[jax.ShapeDtypeStruct((_N, _D), jnp.float32),
               jax.ShapeDtypeStruct((_N, 1), jnp.float32)],
)

_tc2 = pl.pallas_call(
    _tc2_body,
    grid=(_GRID,),
    in_specs=[_s_spec, _row_spec, _col_spec, _b_spec, _w_spec],
    out_specs=[_row_spec],
    out_shape=[jax.ShapeDtypeStruct((_N, _D), jnp.float32)],
)

_tc3 = pl.pallas_call(
    _tc3_body,
    grid=(_GRID,),
    in_specs=[_s_spec, _row_spec, _col_spec, _b_spec,
              pl.BlockSpec((64, _D), lambda i: (0, 0)),
              pl.BlockSpec((64, 16), lambda i: (0, 0))],
    out_specs=[pl.BlockSpec((_BLK, 10), lambda i: (i, 0)),
               pl.BlockSpec((_BLK, 10), lambda i: (i, 0)),
               _row_spec,
               pl.BlockSpec((_BLK, 50), lambda i: (i, 0))],
    out_shape=[jax.ShapeDtypeStruct((_N, 10), jnp.float32),
               jax.ShapeDtypeStruct((_N, 10), jnp.float32),
               jax.ShapeDtypeStruct((_N, _D), jnp.float32),
               jax.ShapeDtypeStruct((_N, 50), jnp.float32)],
)


def kernel(x, edge_index, W1, b1, W2, b2, W3, b3, proto, W_last):
    e0 = edge_index[0].astype(jnp.int32).reshape(_NW, _EW)
    e1 = edge_index[1].astype(jnp.int32).reshape(_NW, _EW)
    pad_src = jnp.broadcast_to((jnp.arange(_EPAD, dtype=jnp.int32) * 89) % _N,
                               (_NW, _EPAD))
    pad_dst = jnp.broadcast_to(
        _N + 16 + (jnp.arange(_EPAD, dtype=jnp.int32) % 208), (_NW, _EPAD))
    src = jnp.concatenate([e0, pad_src], axis=1).reshape(_NW, 1, _EWP)
    dst = jnp.concatenate([e1, pad_dst], axis=1).reshape(_NW, _NCH, _CH)

    _sc_degree, _sc_scatter = _sc_kernels()
    deg2 = _sc_degree(dst)
    deg0 = deg2[0, 0, :_N].reshape(_N, 1)
    deg1 = deg2[1, 0, :_N].reshape(_N, 1)

    b1r = b1.reshape(1, _D)
    b2r = b2.reshape(1, _D)
    b3r = b3.reshape(1, _D)
    proto_pad = jnp.zeros((64, _D), jnp.float32).at[:50].set(proto)
    wlt_pad = jnp.zeros((64, 16), jnp.float32).at[:50, :10].set(W_last.T)

    g1, dinv = _tc1(x, W1, deg0, deg1)

    s = _sc_scatter(g1, src, dst)
    g2, = _tc2(s, g1, dinv, b1r, W2)

    s = _sc_scatter(g2, src, dst)
    g3, = _tc2(s, g2, dinv, b2r, W3)

    s = _sc_scatter(g3, src, dst)
    logits, probs, emb, dist = _tc3(s, g3, dinv, b3r, proto_pad, wlt_pad)
    return (logits, probs, emb, dist)


# speedup vs baseline: 22.1828x; 1.1026x over previous
import functools

import jax
import jax.numpy as jnp
from jax import lax
from jax.experimental import pallas as pl
from jax.experimental.pallas import tpu as pltpu
from jax.experimental.pallas import tpu_sc as plsc

_N = 10000
_D = 128
_E = 320000
_NC = 2
_NS = 16
_NW = _NC * _NS
_EW = _E // _NW
_CH = 128
_EWP = 10240
_EPAD = _EWP - _EW
_NCH = _EWP // _CH
_SLAB = 8
_NSLAB = _NCH // _SLAB
_DEG_PAD = 10240
_DEG_W = _DEG_PAD // _NS
_NPADR = 10240
_ROWS_W = _NPADR // _NS
_ZR = 128

@functools.cache
def _sc_kernels():
    mesh = plsc.VectorSubcoreMesh(core_axis_name="c", subcore_axis_name="s")

    sc_degree = functools.partial(
        pl.kernel,
        out_type=jax.ShapeDtypeStruct((_NC, 1, _DEG_PAD), jnp.float32),
        mesh=mesh,
        scratch_types=[
            pltpu.VMEM((_NSLAB, _SLAB, _CH), jnp.int32),
            pltpu.VMEM((_CH,), jnp.float32),
            pltpu.VMEM((_DEG_W,), jnp.float32),
            pltpu.VMEM_SHARED((_DEG_PAD,), jnp.float32),
            pltpu.SemaphoreType.DMA,
        ],
    )(_sc_degree_body)

    sc_scatter = functools.partial(
        pl.kernel,
        out_type=jax.ShapeDtypeStruct((_NC, _NPADR, _D), jnp.float32),
        mesh=mesh,
        scratch_types=[
            pltpu.VMEM((_EWP,), jnp.int32),
            pltpu.VMEM((_SLAB, _CH), jnp.int32),
            pltpu.VMEM((_SLAB, _CH), jnp.int32),
            pltpu.VMEM((_CH, _D), jnp.float32),
            pltpu.VMEM((_CH, _D), jnp.float32),
            pltpu.VMEM_SHARED((_NPADR, _D), jnp.float32),
            pltpu.SemaphoreType.DMA,
            pltpu.SemaphoreType.DMA,
            pltpu.SemaphoreType.DMA,
            pltpu.SemaphoreType.DMA,
            pltpu.SemaphoreType.DMA,
            pltpu.SemaphoreType.DMA,
            pltpu.SemaphoreType.DMA,
        ],
    )(_sc_scatter_body)

    return sc_degree, sc_scatter


def _sc_degree_body(dst_hbm, deg_hbm, idx_v, ones_v, z_v, acc_sh, sem):
    cid = lax.axis_index("c")
    sid = lax.axis_index("s")
    wid = cid * _NS + sid

    for t in range(0, _CH - 15, 16):
        ones_v[pl.ds(t, 16)] = jnp.ones((16,), jnp.float32)
    if _CH % 16:
        ones_v[pl.ds(_CH - 16, 16)] = jnp.ones((16,), jnp.float32)

    def zfill(t, carry):
        z_v[pl.ds(t * 16, 16)] = jnp.zeros((16,), jnp.float32)
        return carry

    lax.fori_loop(0, _DEG_W // 16, zfill, 0)
    pltpu.sync_copy(z_v, acc_sh.at[pl.ds(sid * _DEG_W, _DEG_W)])
    plsc.subcore_barrier()

    pltpu.sync_copy(dst_hbm.at[wid], idx_v)

    def body(q, carry):
        for r in range(_SLAB):
            pltpu.sync_copy(ones_v, acc_sh.at[idx_v.at[q, r]], add=True)
        return carry

    lax.fori_loop(0, _NSLAB, body, 0)
    plsc.subcore_barrier()
    pltpu.sync_copy(acc_sh.at[pl.ds(sid * _DEG_W, _DEG_W)],
                    deg_hbm.at[cid, 0, pl.ds(sid * _DEG_W, _DEG_W)])


def _sc_scatter_body(g_hbm, src_hbm, dst_hbm, out_hbm,
                     sidx_v, dsl0, dsl1, rows_a, rows_b, acc_sh,
                     isem, dsa, dsb, gsa, gsb, ssa, ssb):
    cid = lax.axis_index("c")
    sid = lax.axis_index("s")
    wid = cid * _NS + sid

    ci_s = pltpu.make_async_copy(src_hbm.at[wid, 0], sidx_v, isem)
    ci_s.start()
    pltpu.async_copy(dst_hbm.at[wid, 0], dsl0, dsa)
    pltpu.async_copy(dst_hbm.at[wid, 1], dsl1, dsb)

    def zfill(r, carry):
        for t in range(_D // 16):
            rows_a[r, pl.ds(t * 16, 16)] = jnp.zeros((16,), jnp.float32)
        return carry

    lax.fori_loop(0, _CH, zfill, 0)
    for t in range(_ROWS_W // _CH):
        pltpu.sync_copy(rows_a,
                        acc_sh.at[pl.ds(sid * _ROWS_W + t * _CH, _CH)])
    ci_s.wait()
    plsc.subcore_barrier()

    def g_start(j, buf, sem):
        pltpu.async_copy(g_hbm.at[sidx_v.at[pl.ds(j * _CH, _CH)]], buf, sem)

    def g_wait(buf, sem):
        pltpu.make_async_copy(g_hbm.at[sidx_v.at[pl.ds(0, _CH)]],
                              buf, sem).wait()

    def s_start(buf, idxrow, sem):
        pltpu.async_copy(buf, acc_sh.at[idxrow], sem, add=True)

    def s_wait(buf, idxrow, sem):
        pltpu.make_async_copy(buf, acc_sh.at[idxrow], sem).wait()

    def d_wait(dsl, sem):
        pltpu.make_async_copy(dst_hbm.at[wid, 0], dsl, sem).wait()

    g_start(0, rows_a, gsa)
    g_start(1, rows_b, gsb)

    def body(i, carry):
        base = 2 * _SLAB * i
        for p, (dsl, dsem) in enumerate(((dsl0, dsa), (dsl1, dsb))):
            d_wait(dsl, dsem)
            for kk in range(0, _SLAB, 2):
                j = base + _SLAB * p + kk
                ia = dsl.at[kk]
                ib = dsl.at[kk + 1]
                g_wait(rows_a, gsa)
                s_start(rows_a, ia, ssa)
                g_wait(rows_b, gsb)
                s_start(rows_b, ib, ssb)
                s_wait(rows_a, ia, ssa)

                @pl.when(j + 2 < _NCH)
                def _():
                    g_start(j + 2, rows_a, gsa)

                s_wait(rows_b, ib, ssb)

                @pl.when(j + 3 < _NCH)
                def _():
                    g_start(j + 3, rows_b, gsb)

            nxt = 2 * (i + 1) + p

            @pl.when(nxt < _NSLAB)
            def _():
                pltpu.async_copy(dst_hbm.at[wid, nxt], dsl, dsem)

        return carry

    lax.fori_loop(0, _NSLAB // 2, body, 0)
    plsc.subcore_barrier()
    pltpu.sync_copy(acc_sh.at[pl.ds(sid * _ROWS_W, _ROWS_W)],
                    out_hbm.at[cid, pl.ds(sid * _ROWS_W, _ROWS_W)])


def _tc1_body(x_ref, w_ref, d0_ref, d1_ref, g_ref, dinv_ref):
    deg = d0_ref[...] + d1_ref[...] + 1.0
    dinv = lax.rsqrt(deg)
    h = jnp.dot(x_ref[...], w_ref[...], preferred_element_type=jnp.float32)
    g_ref[...] = h * dinv
    dinv_ref[...] = dinv


def _tc2_body(s_ref, g_ref, dinv_ref, b_ref, w_ref, gn_ref):
    dinv = dinv_ref[...]
    agg = dinv * (s_ref[0] + s_ref[1] + g_ref[...]) + b_ref[...]
    h = jnp.maximum(agg, 0.0)
    hw = jnp.dot(h, w_ref[...], preferred_element_type=jnp.float32)
    gn_ref[...] = hw * dinv


def _tc3_body(s_ref, g_ref, dinv_ref, b_ref, proto_ref, wlt_ref,
              logits_ref, probs_ref, emb_ref, dist_ref):
    agg = dinv_ref[...] * (s_ref[0] + s_ref[1] + g_ref[...]) + b_ref[...]
    emb = jnp.maximum(agg, 0.0)
    proto = proto_ref[...]
    xp = lax.dot_general(emb, proto, (((1,), (1,)), ((), ())),
                         preferred_element_type=jnp.float32)
    psq = jnp.sum(proto * proto, axis=1)[None, :]
    esq = jnp.sum(emb * emb, axis=1, keepdims=True)
    dist = -2.0 * xp + esq + psq
    sim = jnp.log((dist + 1.0) / (dist + 1e-4))
    logits = jnp.dot(sim, wlt_ref[...], preferred_element_type=jnp.float32)
    col = lax.broadcasted_iota(jnp.int32, logits.shape, 1)
    ml = jnp.where(col < 10, logits, -1e30)
    m = jnp.max(ml, axis=1, keepdims=True)
    e = jnp.exp(ml - m)
    p = e / jnp.sum(e, axis=1, keepdims=True)
    logits_ref[...] = logits[:, :10]
    probs_ref[...] = p[:, :10]
    emb_ref[...] = emb
    dist_ref[...] = dist[:, :50]


_BLK = 1000
_GRID = _N // _BLK

_row_spec = pl.BlockSpec((_BLK, _D), lambda i: (i, 0))
_col_spec = pl.BlockSpec((_BLK, 1), lambda i: (i, 0))
_w_spec = pl.BlockSpec((_D, _D), lambda i: (0, 0))
_b_spec = pl.BlockSpec((1, _D), lambda i: (0, 0))
_s_spec = pl.BlockSpec((_NC, _BLK, _D), lambda i: (0, i, 0))

_tc1 = pl.pallas_call(
    _tc1_body,
    grid=(_GRID,),
    in_specs=[_row_spec, _w_spec, _col_spec, _col_spec],
    out_specs=[_row_spec, _col_spec],
    out_shape=[jax.ShapeDtypeStruct((_N, _D), jnp.float32),
               jax.ShapeDtypeStruct((_N, 1), jnp.float32)],
)

_tc2 = pl.pallas_call(
    _tc2_body,
    grid=(_GRID,),
    in_specs=[_s_spec, _row_spec, _col_spec, _b_spec, _w_spec],
    out_specs=[_row_spec],
    out_shape=[jax.ShapeDtypeStruct((_N, _D), jnp.float32)],
)

_tc3 = pl.pallas_call(
    _tc3_body,
    grid=(_GRID,),
    in_specs=[_s_spec, _row_spec, _col_spec, _b_spec,
              pl.BlockSpec((64, _D), lambda i: (0, 0)),
              pl.BlockSpec((64, 16), lambda i: (0, 0))],
    out_specs=[pl.BlockSpec((_BLK, 10), lambda i: (i, 0)),
               pl.BlockSpec((_BLK, 10), lambda i: (i, 0)),
               _row_spec,
               pl.BlockSpec((_BLK, 50), lambda i: (i, 0))],
    out_shape=[jax.ShapeDtypeStruct((_N, 10), jnp.float32),
               jax.ShapeDtypeStruct((_N, 10), jnp.float32),
               jax.ShapeDtypeStruct((_N, _D), jnp.float32),
               jax.ShapeDtypeStruct((_N, 50), jnp.float32)],
)


def kernel(x, edge_index, W1, b1, W2, b2, W3, b3, proto, W_last):
    e0 = edge_index[0].astype(jnp.int32).reshape(_NW, _EW)
    e1 = edge_index[1].astype(jnp.int32).reshape(_NW, _EW)
    pad_src = jnp.broadcast_to((jnp.arange(_EPAD, dtype=jnp.int32) * 89) % _N,
                               (_NW, _EPAD))
    pad_dst = jnp.broadcast_to(
        _N + 16 + (jnp.arange(_EPAD, dtype=jnp.int32) % 224), (_NW, _EPAD))
    src = jnp.concatenate([e0, pad_src], axis=1).reshape(_NW, 1, _EWP)
    dst = jnp.concatenate([e1, pad_dst], axis=1).reshape(
        _NW, _NSLAB, _SLAB, _CH)

    _sc_degree, _sc_scatter = _sc_kernels()
    deg2 = _sc_degree(dst)
    deg0 = deg2[0, 0, :_N].reshape(_N, 1)
    deg1 = deg2[1, 0, :_N].reshape(_N, 1)

    b1r = b1.reshape(1, _D)
    b2r = b2.reshape(1, _D)
    b3r = b3.reshape(1, _D)
    proto_pad = jnp.zeros((64, _D), jnp.float32).at[:50].set(proto)
    wlt_pad = jnp.zeros((64, 16), jnp.float32).at[:50, :10].set(W_last.T)

    g1, dinv = _tc1(x, W1, deg0, deg1)

    s = _sc_scatter(g1, src, dst)
    g2, = _tc2(s, g1, dinv, b1r, W2)

    s = _sc_scatter(g2, src, dst)
    g3, = _tc2(s, g2, dinv, b2r, W3)

    s = _sc_scatter(g3, src, dst)
    logits, probs, emb, dist = _tc3(s, g3, dinv, b3r, proto_pad, wlt_pad)
    return (logits, probs, emb, dist)


# speedup vs baseline: 27.5568x; 1.2423x over previous
import functools

import jax
import jax.numpy as jnp
from jax import lax
from jax.experimental import pallas as pl
from jax.experimental.pallas import tpu as pltpu
from jax.experimental.pallas import tpu_sc as plsc

_N = 10000
_D = 128
_E = 320000
_NC = 2
_NS = 16
_NW = _NC * _NS
_EW = _E // _NW
_CH = 64
_EWP = 10240
_EPAD = _EWP - _EW
_NCH = _EWP // _CH
_SLAB = 16
_NSLAB = _NCH // _SLAB
_DEG_PAD = 10240
_DEG_W = _DEG_PAD // _NS
_NPADR = 10240
_ROWS_W = _NPADR // _NS
_ZR = 128

@functools.cache
def _sc_kernels():
    mesh = plsc.VectorSubcoreMesh(core_axis_name="c", subcore_axis_name="s")

    sc_degree = functools.partial(
        pl.kernel,
        out_type=jax.ShapeDtypeStruct((_NC, 1, _DEG_PAD), jnp.float32),
        mesh=mesh,
        scratch_types=[
            pltpu.VMEM((_NSLAB, _SLAB, _CH), jnp.int32),
            pltpu.VMEM((_CH,), jnp.float32),
            pltpu.VMEM((_DEG_W,), jnp.float32),
            pltpu.VMEM_SHARED((_DEG_PAD,), jnp.float32),
            pltpu.SemaphoreType.DMA,
        ],
    )(_sc_degree_body)

    sc_scatter = functools.partial(
        pl.kernel,
        out_type=jax.ShapeDtypeStruct((_NC, _NPADR, _D), jnp.float32),
        mesh=mesh,
        scratch_types=[
            pltpu.VMEM((_EWP,), jnp.int32),
            pltpu.VMEM((_SLAB, _CH), jnp.int32),
            pltpu.VMEM((_SLAB, _CH), jnp.int32),
            pltpu.VMEM((_CH, _D), jnp.float32),
            pltpu.VMEM((_CH, _D), jnp.float32),
            pltpu.VMEM((_CH, _D), jnp.float32),
            pltpu.VMEM((_CH, _D), jnp.float32),
            pltpu.VMEM_SHARED((_NPADR, _D), jnp.float32),
            pltpu.SemaphoreType.DMA,
            pltpu.SemaphoreType.DMA,
            pltpu.SemaphoreType.DMA,
            pltpu.SemaphoreType.DMA,
            pltpu.SemaphoreType.DMA,
            pltpu.SemaphoreType.DMA,
            pltpu.SemaphoreType.DMA,
            pltpu.SemaphoreType.DMA,
            pltpu.SemaphoreType.DMA,
            pltpu.SemaphoreType.DMA,
            pltpu.SemaphoreType.DMA,
        ],
    )(_sc_scatter_body)

    return sc_degree, sc_scatter


def _sc_degree_body(dst_hbm, deg_hbm, idx_v, ones_v, z_v, acc_sh, sem):
    cid = lax.axis_index("c")
    sid = lax.axis_index("s")
    wid = cid * _NS + sid

    for t in range(0, _CH - 15, 16):
        ones_v[pl.ds(t, 16)] = jnp.ones((16,), jnp.float32)
    if _CH % 16:
        ones_v[pl.ds(_CH - 16, 16)] = jnp.ones((16,), jnp.float32)

    def zfill(t, carry):
        z_v[pl.ds(t * 16, 16)] = jnp.zeros((16,), jnp.float32)
        return carry

    lax.fori_loop(0, _DEG_W // 16, zfill, 0)
    pltpu.sync_copy(z_v, acc_sh.at[pl.ds(sid * _DEG_W, _DEG_W)])
    plsc.subcore_barrier()

    pltpu.sync_copy(dst_hbm.at[wid], idx_v)

    def body(q, carry):
        for r in range(_SLAB):
            pltpu.sync_copy(ones_v, acc_sh.at[idx_v.at[q, r]], add=True)
        return carry

    lax.fori_loop(0, _NSLAB, body, 0)
    plsc.subcore_barrier()
    pltpu.sync_copy(acc_sh.at[pl.ds(sid * _DEG_W, _DEG_W)],
                    deg_hbm.at[cid, 0, pl.ds(sid * _DEG_W, _DEG_W)])


def _sc_scatter_body(g_hbm, src_hbm, dst_hbm, out_hbm,
                     sidx_v, dsl0, dsl1, rows_a, rows_b, rows_c, rows_d,
                     acc_sh, isem, dsa, dsb,
                     gsa, gsb, gsc, gsd, ssa, ssb, ssc, ssd):
    cid = lax.axis_index("c")
    sid = lax.axis_index("s")
    wid = cid * _NS + sid

    ci_s = pltpu.make_async_copy(src_hbm.at[wid, 0], sidx_v, isem)
    ci_s.start()
    pltpu.async_copy(dst_hbm.at[wid, 0], dsl0, dsa)
    pltpu.async_copy(dst_hbm.at[wid, 1], dsl1, dsb)

    def zfill(r, carry):
        for t in range(_D // 16):
            rows_a[r, pl.ds(t * 16, 16)] = jnp.zeros((16,), jnp.float32)
        return carry

    lax.fori_loop(0, _CH, zfill, 0)
    for t in range(_ROWS_W // _CH):
        pltpu.sync_copy(rows_a,
                        acc_sh.at[pl.ds(sid * _ROWS_W + t * _CH, _CH)])
    ci_s.wait()
    plsc.subcore_barrier()

    def g_start(j, buf, sem):
        pltpu.async_copy(g_hbm.at[sidx_v.at[pl.ds(j * _CH, _CH)]], buf, sem)

    def g_wait(buf, sem):
        pltpu.make_async_copy(g_hbm.at[sidx_v.at[pl.ds(0, _CH)]],
                              buf, sem).wait()

    def s_start(buf, idxrow, sem):
        pltpu.async_copy(buf, acc_sh.at[idxrow], sem, add=True)

    def s_wait(buf, idxrow, sem):
        pltpu.make_async_copy(buf, acc_sh.at[idxrow], sem).wait()

    def d_wait(dsl, sem):
        pltpu.make_async_copy(dst_hbm.at[wid, 0], dsl, sem).wait()

    bufs = ((rows_a, gsa, ssa), (rows_b, gsb, ssb),
            (rows_c, gsc, ssc), (rows_d, gsd, ssd))
    for q in range(4):
        g_start(q, bufs[q][0], bufs[q][1])

    def body(i, carry):
        base = 2 * _SLAB * i
        for p, (dsl, dsem) in enumerate(((dsl0, dsa), (dsl1, dsb))):
            d_wait(dsl, dsem)
            for kk in range(0, _SLAB, 4):
                j = base + _SLAB * p + kk
                idxrows = [dsl.at[kk + q] for q in range(4)]
                g_wait(rows_a, gsa)
                s_start(rows_a, idxrows[0], ssa)
                g_wait(rows_b, gsb)
                s_start(rows_b, idxrows[1], ssb)
                s_wait(rows_a, idxrows[0], ssa)

                @pl.when(j + 4 < _NCH)
                def _():
                    g_start(j + 4, rows_a, gsa)

                g_wait(rows_c, gsc)
                s_start(rows_c, idxrows[2], ssc)
                s_wait(rows_b, idxrows[1], ssb)

                @pl.when(j + 5 < _NCH)
                def _():
                    g_start(j + 5, rows_b, gsb)

                g_wait(rows_d, gsd)
                s_start(rows_d, idxrows[3], ssd)
                s_wait(rows_c, idxrows[2], ssc)

                @pl.when(j + 6 < _NCH)
                def _():
                    g_start(j + 6, rows_c, gsc)

                s_wait(rows_d, idxrows[3], ssd)

                @pl.when(j + 7 < _NCH)
                def _():
                    g_start(j + 7, rows_d, gsd)

            nxt = 2 * (i + 1) + p

            @pl.when(nxt < _NSLAB)
            def _():
                pltpu.async_copy(dst_hbm.at[wid, nxt], dsl, dsem)

        return carry

    lax.fori_loop(0, _NSLAB // 2, body, 0)
    plsc.subcore_barrier()
    pltpu.sync_copy(acc_sh.at[pl.ds(sid * _ROWS_W, _ROWS_W)],
                    out_hbm.at[cid, pl.ds(sid * _ROWS_W, _ROWS_W)])


def _tc1_body(x_ref, w_ref, d0_ref, d1_ref, g_ref, dinv_ref):
    deg = d0_ref[...] + d1_ref[...] + 1.0
    dinv = lax.rsqrt(deg)
    h = jnp.dot(x_ref[...], w_ref[...], preferred_element_type=jnp.float32)
    g_ref[...] = h * dinv
    dinv_ref[...] = dinv


def _tc2_body(s_ref, g_ref, dinv_ref, b_ref, w_ref, gn_ref):
    dinv = dinv_ref[...]
    agg = dinv * (s_ref[0] + s_ref[1] + g_ref[...]) + b_ref[...]
    h = jnp.maximum(agg, 0.0)
    hw = jnp.dot(h, w_ref[...], preferred_element_type=jnp.float32)
    gn_ref[...] = hw * dinv


def _tc3_body(s_ref, g_ref, dinv_ref, b_ref, proto_ref, wlt_ref,
              logits_ref, probs_ref, emb_ref, dist_ref):
    agg = dinv_ref[...] * (s_ref[0] + s_ref[1] + g_ref[...]) + b_ref[...]
    emb = jnp.maximum(agg, 0.0)
    proto = proto_ref[...]
    xp = lax.dot_general(emb, proto, (((1,), (1,)), ((), ())),
                         preferred_element_type=jnp.float32)
    psq = jnp.sum(proto * proto, axis=1)[None, :]
    esq = jnp.sum(emb * emb, axis=1, keepdims=True)
    dist = -2.0 * xp + esq + psq
    sim = jnp.log((dist + 1.0) / (dist + 1e-4))
    logits = jnp.dot(sim, wlt_ref[...], preferred_element_type=jnp.float32)
    col = lax.broadcasted_iota(jnp.int32, logits.shape, 1)
    ml = jnp.where(col < 10, logits, -1e30)
    m = jnp.max(ml, axis=1, keepdims=True)
    e = jnp.exp(ml - m)
    p = e / jnp.sum(e, axis=1, keepdims=True)
    logits_ref[...] = logits[:, :10]
    probs_ref[...] = p[:, :10]
    emb_ref[...] = emb
    dist_ref[...] = dist[:, :50]


_BLK = 1000
_GRID = _N // _BLK

_row_spec = pl.BlockSpec((_BLK, _D), lambda i: (i, 0))
_col_spec = pl.BlockSpec((_BLK, 1), lambda i: (i, 0))
_w_spec = pl.BlockSpec((_D, _D), lambda i: (0, 0))
_b_spec = pl.BlockSpec((1, _D), lambda i: (0, 0))
_s_spec = pl.BlockSpec((_NC, _BLK, _D), lambda i: (0, i, 0))

_tc1 = pl.pallas_call(
    _tc1_body,
    grid=(_GRID,),
    in_specs=[_row_spec, _w_spec, _col_spec, _col_spec],
    out_specs=[_row_spec, _col_spec],
    out_shape=[jax.ShapeDtypeStruct((_N, _D), jnp.float32),
               jax.ShapeDtypeStruct((_N, 1), jnp.float32)],
)

_tc2 = pl.pallas_call(
    _tc2_body,
    grid=(_GRID,),
    in_specs=[_s_spec, _row_spec, _col_spec, _b_spec, _w_spec],
    out_specs=[_row_spec],
    out_shape=[jax.ShapeDtypeStruct((_N, _D), jnp.float32)],
)

_tc3 = pl.pallas_call(
    _tc3_body,
    grid=(_GRID,),
    in_specs=[_s_spec, _row_spec, _col_spec, _b_spec,
              pl.BlockSpec((64, _D), lambda i: (0, 0)),
              pl.BlockSpec((64, 16), lambda i: (0, 0))],
    out_specs=[pl.BlockSpec((_BLK, 10), lambda i: (i, 0)),
               pl.BlockSpec((_BLK, 10), lambda i: (i, 0)),
               _row_spec,
               pl.BlockSpec((_BLK, 50), lambda i: (i, 0))],
    out_shape=[jax.ShapeDtypeStruct((_N, 10), jnp.float32),
               jax.ShapeDtypeStruct((_N, 10), jnp.float32),
               jax.ShapeDtypeStruct((_N, _D), jnp.float32),
               jax.ShapeDtypeStruct((_N, 50), jnp.float32)],
)


def kernel(x, edge_index, W1, b1, W2, b2, W3, b3, proto, W_last):
    e0 = edge_index[0].astype(jnp.int32).reshape(_NW, _EW)
    e1 = edge_index[1].astype(jnp.int32).reshape(_NW, _EW)
    pad_src = jnp.broadcast_to((jnp.arange(_EPAD, dtype=jnp.int32) * 89) % _N,
                               (_NW, _EPAD))
    pad_dst = jnp.broadcast_to(
        _N + 16 + (jnp.arange(_EPAD, dtype=jnp.int32) % 224), (_NW, _EPAD))
    src = jnp.concatenate([e0, pad_src], axis=1).reshape(_NW, 1, _EWP)
    dst = jnp.concatenate([e1, pad_dst], axis=1).reshape(
        _NW, _NSLAB, _SLAB, _CH)

    _sc_degree, _sc_scatter = _sc_kernels()
    deg2 = _sc_degree(dst)
    deg0 = deg2[0, 0, :_N].reshape(_N, 1)
    deg1 = deg2[1, 0, :_N].reshape(_N, 1)

    b1r = b1.reshape(1, _D)
    b2r = b2.reshape(1, _D)
    b3r = b3.reshape(1, _D)
    proto_pad = jnp.zeros((64, _D), jnp.float32).at[:50].set(proto)
    wlt_pad = jnp.zeros((64, 16), jnp.float32).at[:50, :10].set(W_last.T)

    g1, dinv = _tc1(x, W1, deg0, deg1)

    s = _sc_scatter(g1, src, dst)
    g2, = _tc2(s, g1, dinv, b1r, W2)

    s = _sc_scatter(g2, src, dst)
    g3, = _tc2(s, g2, dinv, b2r, W3)

    s = _sc_scatter(g3, src, dst)
    logits, probs, emb, dist = _tc3(s, g3, dinv, b3r, proto_pad, wlt_pad)
    return (logits, probs, emb, dist)


# speedup vs baseline: 28.1258x; 1.0206x over previous
import functools

import jax
import jax.numpy as jnp
from jax import lax
from jax.experimental import pallas as pl
from jax.experimental.pallas import tpu as pltpu
from jax.experimental.pallas import tpu_sc as plsc

_N = 10000
_D = 128
_E = 320000
_NC = 2
_NS = 16
_NW = _NC * _NS
_EW = _E // _NW
_CH = 64
_EWP = 10240
_EPAD = _EWP - _EW
_NCH = _EWP // _CH
_SLAB = 16
_NSLAB = _NCH // _SLAB
_DEG_PAD = 10240
_DEG_W = _DEG_PAD // _NS
_NPADR = 10240
_ROWS_W = _NPADR // _NS
_ZR = 128

@functools.cache
def _sc_kernels():
    mesh = plsc.VectorSubcoreMesh(core_axis_name="c", subcore_axis_name="s")

    sc_degree = functools.partial(
        pl.kernel,
        out_type=jax.ShapeDtypeStruct((_NC, 1, _DEG_PAD), jnp.float32),
        mesh=mesh,
        scratch_types=[
            pltpu.VMEM((_NSLAB, _SLAB, _CH), jnp.int32),
            pltpu.VMEM((_CH,), jnp.float32),
            pltpu.VMEM((_DEG_W,), jnp.float32),
            pltpu.VMEM_SHARED((_DEG_PAD,), jnp.float32),
            pltpu.SemaphoreType.DMA,
            pltpu.SemaphoreType.DMA,
            pltpu.SemaphoreType.DMA,
            pltpu.SemaphoreType.DMA,
        ],
    )(_sc_degree_body)

    sc_scatter = functools.partial(
        pl.kernel,
        out_type=jax.ShapeDtypeStruct((_NC, _NPADR, _D), jnp.float32),
        mesh=mesh,
        scratch_types=[
            pltpu.VMEM((_EWP,), jnp.int32),
            pltpu.VMEM((_SLAB, _CH), jnp.int32),
            pltpu.VMEM((_SLAB, _CH), jnp.int32),
            pltpu.VMEM((_CH, _D), jnp.float32),
            pltpu.VMEM((_CH, _D), jnp.float32),
            pltpu.VMEM((_CH, _D), jnp.float32),
            pltpu.VMEM((_CH, _D), jnp.float32),
            pltpu.VMEM_SHARED((_NPADR, _D), jnp.float32),
            pltpu.SemaphoreType.DMA,
            pltpu.SemaphoreType.DMA,
            pltpu.SemaphoreType.DMA,
            pltpu.SemaphoreType.DMA,
            pltpu.SemaphoreType.DMA,
            pltpu.SemaphoreType.DMA,
            pltpu.SemaphoreType.DMA,
            pltpu.SemaphoreType.DMA,
            pltpu.SemaphoreType.DMA,
            pltpu.SemaphoreType.DMA,
            pltpu.SemaphoreType.DMA,
        ],
    )(_sc_scatter_body)

    return sc_degree, sc_scatter


def _sc_degree_body(dst_hbm, deg_hbm, idx_v, ones_v, z_v, acc_sh,
                    sem, s0, s1, s2):
    cid = lax.axis_index("c")
    sid = lax.axis_index("s")
    wid = cid * _NS + sid

    for t in range(0, _CH - 15, 16):
        ones_v[pl.ds(t, 16)] = jnp.ones((16,), jnp.float32)
    if _CH % 16:
        ones_v[pl.ds(_CH - 16, 16)] = jnp.ones((16,), jnp.float32)

    def zfill(t, carry):
        z_v[pl.ds(t * 16, 16)] = jnp.zeros((16,), jnp.float32)
        return carry

    lax.fori_loop(0, _DEG_W // 16, zfill, 0)
    pltpu.sync_copy(z_v, acc_sh.at[pl.ds(sid * _DEG_W, _DEG_W)])
    plsc.subcore_barrier()

    pltpu.sync_copy(dst_hbm.at[wid], idx_v)
    sems = (sem, s0, s1, s2)

    def h_start(q, r, k):
        pltpu.async_copy(ones_v, acc_sh.at[idx_v.at[q, r]], sems[k],
                         add=True)

    def h_wait(k):
        pltpu.make_async_copy(ones_v, acc_sh.at[idx_v.at[0, 0]],
                              sems[k]).wait()

    def body(q, carry):
        for r in range(_SLAB):
            if r >= 4:
                h_wait(r % 4)
            else:
                @pl.when(q > 0)
                def _():
                    h_wait(r % 4)
            h_start(q, r, r % 4)
        return carry

    lax.fori_loop(0, _NSLAB, body, 0)
    for k in range(4):
        h_wait(k)
    plsc.subcore_barrier()
    pltpu.sync_copy(acc_sh.at[pl.ds(sid * _DEG_W, _DEG_W)],
                    deg_hbm.at[cid, 0, pl.ds(sid * _DEG_W, _DEG_W)])


def _sc_scatter_body(g_hbm, src_hbm, dst_hbm, out_hbm,
                     sidx_v, dsl0, dsl1, rows_a, rows_b, rows_c, rows_d,
                     acc_sh, isem, dsa, dsb,
                     gsa, gsb, gsc, gsd, ssa, ssb, ssc, ssd):
    cid = lax.axis_index("c")
    sid = lax.axis_index("s")
    wid = cid * _NS + sid

    ci_s = pltpu.make_async_copy(src_hbm.at[wid, 0], sidx_v, isem)
    ci_s.start()
    pltpu.async_copy(dst_hbm.at[wid, 0], dsl0, dsa)
    pltpu.async_copy(dst_hbm.at[wid, 1], dsl1, dsb)

    def zfill(r, carry):
        for t in range(_D // 16):
            rows_a[r, pl.ds(t * 16, 16)] = jnp.zeros((16,), jnp.float32)
        return carry

    ci_s.wait()
    pltpu.async_copy(g_hbm.at[sidx_v.at[pl.ds(1 * _CH, _CH)]], rows_b, gsb)
    pltpu.async_copy(g_hbm.at[sidx_v.at[pl.ds(2 * _CH, _CH)]], rows_c, gsc)
    pltpu.async_copy(g_hbm.at[sidx_v.at[pl.ds(3 * _CH, _CH)]], rows_d, gsd)

    lax.fori_loop(0, _CH, zfill, 0)
    for t in range(_ROWS_W // _CH):
        pltpu.sync_copy(rows_a,
                        acc_sh.at[pl.ds(sid * _ROWS_W + t * _CH, _CH)])
    pltpu.async_copy(g_hbm.at[sidx_v.at[pl.ds(0, _CH)]], rows_a, gsa)
    plsc.subcore_barrier()

    def g_start(j, buf, sem):
        pltpu.async_copy(g_hbm.at[sidx_v.at[pl.ds(j * _CH, _CH)]], buf, sem)

    def g_wait(buf, sem):
        pltpu.make_async_copy(g_hbm.at[sidx_v.at[pl.ds(0, _CH)]],
                              buf, sem).wait()

    def s_start(buf, idxrow, sem):
        pltpu.async_copy(buf, acc_sh.at[idxrow], sem, add=True)

    def s_wait(buf, idxrow, sem):
        pltpu.make_async_copy(buf, acc_sh.at[idxrow], sem).wait()

    def d_wait(dsl, sem):
        pltpu.make_async_copy(dst_hbm.at[wid, 0], dsl, sem).wait()

    def body(i, carry):
        base = 2 * _SLAB * i
        for p, (dsl, dsem) in enumerate(((dsl0, dsa), (dsl1, dsb))):
            d_wait(dsl, dsem)
            for kk in range(0, _SLAB, 4):
                j = base + _SLAB * p + kk
                idxrows = [dsl.at[kk + q] for q in range(4)]
                g_wait(rows_a, gsa)
                s_start(rows_a, idxrows[0], ssa)
                g_wait(rows_b, gsb)
                s_start(rows_b, idxrows[1], ssb)
                s_wait(rows_a, idxrows[0], ssa)

                @pl.when(j + 4 < _NCH)
                def _():
                    g_start(j + 4, rows_a, gsa)

                g_wait(rows_c, gsc)
                s_start(rows_c, idxrows[2], ssc)
                s_wait(rows_b, idxrows[1], ssb)

                @pl.when(j + 5 < _NCH)
                def _():
                    g_start(j + 5, rows_b, gsb)

                g_wait(rows_d, gsd)
                s_start(rows_d, idxrows[3], ssd)
                s_wait(rows_c, idxrows[2], ssc)

                @pl.when(j + 6 < _NCH)
                def _():
                    g_start(j + 6, rows_c, gsc)

                s_wait(rows_d, idxrows[3], ssd)

                @pl.when(j + 7 < _NCH)
                def _():
                    g_start(j + 7, rows_d, gsd)

            nxt = 2 * (i + 1) + p

            @pl.when(nxt < _NSLAB)
            def _():
                pltpu.async_copy(dst_hbm.at[wid, nxt], dsl, dsem)

        return carry

    lax.fori_loop(0, _NSLAB // 2, body, 0)
    plsc.subcore_barrier()
    pltpu.sync_copy(acc_sh.at[pl.ds(sid * _ROWS_W, _ROWS_W)],
                    out_hbm.at[cid, pl.ds(sid * _ROWS_W, _ROWS_W)])


def _tc1_body(x_ref, w_ref, d0_ref, d1_ref, g_ref, dinv_ref):
    deg = d0_ref[...] + d1_ref[...] + 1.0
    dinv = lax.rsqrt(deg)
    h = jnp.dot(x_ref[...], w_ref[...], preferred_element_type=jnp.float32)
    g_ref[...] = h * dinv
    dinv_ref[...] = dinv


def _tc2_body(s_ref, g_ref, dinv_ref, b_ref, w_ref, gn_ref):
    dinv = dinv_ref[...]
    agg = dinv * (s_ref[0] + s_ref[1] + g_ref[...]) + b_ref[...]
    h = jnp.maximum(agg, 0.0)
    hw = jnp.dot(h, w_ref[...], preferred_element_type=jnp.float32)
    gn_ref[...] = hw * dinv


def _tc3_body(s_ref, g_ref, dinv_ref, b_ref, proto_ref, wlt_ref,
              logits_ref, probs_ref, emb_ref, dist_ref):
    agg = dinv_ref[...] * (s_ref[0] + s_ref[1] + g_ref[...]) + b_ref[...]
    emb = jnp.maximum(agg, 0.0)
    proto = proto_ref[...]
    xp = lax.dot_general(emb, proto, (((1,), (1,)), ((), ())),
                         preferred_element_type=jnp.float32)
    psq = jnp.sum(proto * proto, axis=1)[None, :]
    esq = jnp.sum(emb * emb, axis=1, keepdims=True)
    dist = -2.0 * xp + esq + psq
    sim = jnp.log((dist + 1.0) / (dist + 1e-4))
    logits = jnp.dot(sim, wlt_ref[...], preferred_element_type=jnp.float32)
    col = lax.broadcasted_iota(jnp.int32, logits.shape, 1)
    ml = jnp.where(col < 10, logits, -1e30)
    m = jnp.max(ml, axis=1, keepdims=True)
    e = jnp.exp(ml - m)
    p = e / jnp.sum(e, axis=1, keepdims=True)
    logits_ref[...] = logits[:, :10]
    probs_ref[...] = p[:, :10]
    emb_ref[...] = emb
    dist_ref[...] = dist[:, :50]


_BLK = 1000
_GRID = _N // _BLK

_row_spec = pl.BlockSpec((_BLK, _D), lambda i: (i, 0))
_col_spec = pl.BlockSpec((_BLK, 1), lambda i: (i, 0))
_w_spec = pl.BlockSpec((_D, _D), lambda i: (0, 0))
_b_spec = pl.BlockSpec((1, _D), lambda i: (0, 0))
_s_spec = pl.BlockSpec((_NC, _BLK, _D), lambda i: (0, i, 0))

_tc1 = pl.pallas_call(
    _tc1_body,
    grid=(_GRID,),
    in_specs=[_row_spec, _w_spec, _col_spec, _col_spec],
    out_specs=[_row_spec, _col_spec],
    out_shape=[jax.ShapeDtypeStruct((_N, _D), jnp.float32),
               jax.ShapeDtypeStruct((_N, 1), jnp.float32)],
)

_tc2 = pl.pallas_call(
    _tc2_body,
    grid=(_GRID,),
    in_specs=[_s_spec, _row_spec, _col_spec, _b_spec, _w_spec],
    out_specs=[_row_spec],
    out_shape=[jax.ShapeDtypeStruct((_N, _D), jnp.float32)],
)

_tc3 = pl.pallas_call(
    _tc3_body,
    grid=(_GRID,),
    in_specs=[_s_spec, _row_spec, _col_spec, _b_spec,
              pl.BlockSpec((64, _D), lambda i: (0, 0)),
              pl.BlockSpec((64, 16), lambda i: (0, 0))],
    out_specs=[pl.BlockSpec((_BLK, 10), lambda i: (i, 0)),
               pl.BlockSpec((_BLK, 10), lambda i: (i, 0)),
               _row_spec,
               pl.BlockSpec((_BLK, 50), lambda i: (i, 0))],
    out_shape=[jax.ShapeDtypeStruct((_N, 10), jnp.float32),
               jax.ShapeDtypeStruct((_N, 10), jnp.float32),
               jax.ShapeDtypeStruct((_N, _D), jnp.float32),
               jax.ShapeDtypeStruct((_N, 50), jnp.float32)],
)


def kernel(x, edge_index, W1, b1, W2, b2, W3, b3, proto, W_last):
    e0 = edge_index[0].astype(jnp.int32).reshape(_NW, _EW)
    e1 = edge_index[1].astype(jnp.int32).reshape(_NW, _EW)
    pad_src = jnp.broadcast_to((jnp.arange(_EPAD, dtype=jnp.int32) * 89) % _N,
                               (_NW, _EPAD))
    pad_dst = jnp.broadcast_to(
        _N + 16 + (jnp.arange(_EPAD, dtype=jnp.int32) % 224), (_NW, _EPAD))
    src = jnp.concatenate([e0, pad_src], axis=1).reshape(_NW, 1, _EWP)
    dst = jnp.concatenate([e1, pad_dst], axis=1).reshape(
        _NW, _NSLAB, _SLAB, _CH)

    _sc_degree, _sc_scatter = _sc_kernels()
    deg2 = _sc_degree(dst)
    deg0 = deg2[0, 0, :_N].reshape(_N, 1)
    deg1 = deg2[1, 0, :_N].reshape(_N, 1)

    b1r = b1.reshape(1, _D)
    b2r = b2.reshape(1, _D)
    b3r = b3.reshape(1, _D)
    proto_pad = jnp.zeros((64, _D), jnp.float32).at[:50].set(proto)
    wlt_pad = jnp.zeros((64, 16), jnp.float32).at[:50, :10].set(W_last.T)

    g1, dinv = _tc1(x, W1, deg0, deg1)

    s = _sc_scatter(g1, src, dst)
    g2, = _tc2(s, g1, dinv, b1r, W2)

    s = _sc_scatter(g2, src, dst)
    g3, = _tc2(s, g2, dinv, b2r, W3)

    s = _sc_scatter(g3, src, dst)
    logits, probs, emb, dist = _tc3(s, g3, dinv, b3r, proto_pad, wlt_pad)
    return (logits, probs, emb, dist)


# speedup vs baseline: 28.1326x; 1.0002x over previous
import functools

import jax
import jax.numpy as jnp
from jax import lax
from jax.experimental import pallas as pl
from jax.experimental.pallas import tpu as pltpu
from jax.experimental.pallas import tpu_sc as plsc

_N = 10000
_D = 128
_E = 320000
_NC = 2
_NS = 16
_NW = _NC * _NS
_EW = _E // _NW
_CH = 64
_EWP = 10240
_EPAD = _EWP - _EW
_NCH = _EWP // _CH
_SLAB = 16
_NSLAB = _NCH // _SLAB
_DEG_PAD = 10240
_DEG_W = _DEG_PAD // _NS
_NPADR = 10240
_ROWS_W = _NPADR // _NS

@functools.cache
def _sc_kernels():
    mesh = plsc.VectorSubcoreMesh(core_axis_name="c", subcore_axis_name="s")

    sc_degree = functools.partial(
        pl.kernel,
        out_type=jax.ShapeDtypeStruct((_NC, 1, _DEG_PAD), jnp.float32),
        mesh=mesh,
        scratch_types=[
            pltpu.VMEM((_NSLAB, _SLAB, _CH), jnp.int32),
            pltpu.VMEM((_CH,), jnp.float32),
            pltpu.VMEM((_DEG_W,), jnp.float32),
            pltpu.VMEM_SHARED((_DEG_PAD,), jnp.float32),
            pltpu.SemaphoreType.DMA,
            pltpu.SemaphoreType.DMA,
            pltpu.SemaphoreType.DMA,
            pltpu.SemaphoreType.DMA,
        ],
    )(_sc_degree_body)

    sc_scatter = functools.partial(
        pl.kernel,
        out_type=jax.ShapeDtypeStruct((_NC, _NPADR, _D), jnp.float32),
        mesh=mesh,
        scratch_types=[
            pltpu.VMEM((_EWP,), jnp.int32),
            pltpu.VMEM((_SLAB, _CH), jnp.int32),
            pltpu.VMEM((_SLAB, _CH), jnp.int32),
            pltpu.VMEM((_CH, _D), jnp.float32),
            pltpu.VMEM((_CH, _D), jnp.float32),
            pltpu.VMEM((_CH, _D), jnp.float32),
            pltpu.VMEM((_CH, _D), jnp.float32),
            pltpu.VMEM_SHARED((_NPADR, _D), jnp.float32),
            pltpu.SemaphoreType.DMA,
            pltpu.SemaphoreType.DMA,
            pltpu.SemaphoreType.DMA,
            pltpu.SemaphoreType.DMA,
            pltpu.SemaphoreType.DMA,
            pltpu.SemaphoreType.DMA,
            pltpu.SemaphoreType.DMA,
            pltpu.SemaphoreType.DMA,
            pltpu.SemaphoreType.DMA,
            pltpu.SemaphoreType.DMA,
            pltpu.SemaphoreType.DMA,
        ],
    )(_sc_scatter_body)

    return sc_degree, sc_scatter


def _sc_degree_body(dst_hbm, deg_hbm, idx_v, ones_v, z_v, acc_sh,
                    sem, s0, s1, s2):
    cid = lax.axis_index("c")
    sid = lax.axis_index("s")
    wid = cid * _NS + sid

    for t in range(0, _CH - 15, 16):
        ones_v[pl.ds(t, 16)] = jnp.ones((16,), jnp.float32)
    if _CH % 16:
        ones_v[pl.ds(_CH - 16, 16)] = jnp.ones((16,), jnp.float32)

    def zfill(t, carry):
        z_v[pl.ds(t * 16, 16)] = jnp.zeros((16,), jnp.float32)
        return carry

    lax.fori_loop(0, _DEG_W // 16, zfill, 0)
    pltpu.sync_copy(z_v, acc_sh.at[pl.ds(sid * _DEG_W, _DEG_W)])
    plsc.subcore_barrier()

    pltpu.sync_copy(dst_hbm.at[wid], idx_v)
    sems = (sem, s0, s1, s2)

    def h_start(q, r, k):
        pltpu.async_copy(ones_v, acc_sh.at[idx_v.at[q, r]], sems[k],
                         add=True)

    def h_wait(k):
        pltpu.make_async_copy(ones_v, acc_sh.at[idx_v.at[0, 0]],
                              sems[k]).wait()

    def body(q, carry):
        for r in range(_SLAB):
            if r >= 4:
                h_wait(r % 4)
            else:
                @pl.when(q > 0)
                def _():
                    h_wait(r % 4)
            h_start(q, r, r % 4)
        return carry

    lax.fori_loop(0, _NSLAB, body, 0)
    for k in range(4):
        h_wait(k)
    plsc.subcore_barrier()
    pltpu.sync_copy(acc_sh.at[pl.ds(sid * _DEG_W, _DEG_W)],
                    deg_hbm.at[cid, 0, pl.ds(sid * _DEG_W, _DEG_W)])


def _sc_scatter_body(g_hbm, src_hbm, dst_hbm, out_hbm,
                     sidx_v, dsl0, dsl1, rows_a, rows_b, rows_c, rows_d,
                     acc_sh, isem, dsa, dsb,
                     gsa, gsb, gsc, gsd, ssa, ssb, ssc, ssd):
    cid = lax.axis_index("c")
    sid = lax.axis_index("s")
    wid = cid * _NS + sid

    ci_s = pltpu.make_async_copy(src_hbm.at[wid, 0], sidx_v, isem)
    ci_s.start()
    pltpu.async_copy(dst_hbm.at[wid, 0], dsl0, dsa)
    pltpu.async_copy(dst_hbm.at[wid, 1], dsl1, dsb)

    def zfill(r, carry):
        for t in range(_D // 16):
            rows_a[r, pl.ds(t * 16, 16)] = jnp.zeros((16,), jnp.float32)
        return carry

    ci_s.wait()
    pltpu.async_copy(g_hbm.at[sidx_v.at[pl.ds(1 * _CH, _CH)]], rows_b, gsb)
    pltpu.async_copy(g_hbm.at[sidx_v.at[pl.ds(2 * _CH, _CH)]], rows_c, gsc)
    pltpu.async_copy(g_hbm.at[sidx_v.at[pl.ds(3 * _CH, _CH)]], rows_d, gsd)

    lax.fori_loop(0, _CH, zfill, 0)
    for t in range(_ROWS_W // _CH):
        pltpu.sync_copy(rows_a,
                        acc_sh.at[pl.ds(sid * _ROWS_W + t * _CH, _CH)])
    pltpu.async_copy(g_hbm.at[sidx_v.at[pl.ds(0, _CH)]], rows_a, gsa)
    plsc.subcore_barrier()

    def g_start(j, buf, sem):
        pltpu.async_copy(g_hbm.at[sidx_v.at[pl.ds(j * _CH, _CH)]], buf, sem)

    def g_wait(buf, sem):
        pltpu.make_async_copy(g_hbm.at[sidx_v.at[pl.ds(0, _CH)]],
                              buf, sem).wait()

    def s_start(buf, idxrow, sem):
        pltpu.async_copy(buf, acc_sh.at[idxrow], sem, add=True)

    def s_wait(buf, idxrow, sem):
        pltpu.make_async_copy(buf, acc_sh.at[idxrow], sem).wait()

    def d_wait(dsl, sem):
        pltpu.make_async_copy(dst_hbm.at[wid, 0], dsl, sem).wait()

    def body(i, carry):
        base = 2 * _SLAB * i
        for p, (dsl, dsem) in enumerate(((dsl0, dsa), (dsl1, dsb))):
            d_wait(dsl, dsem)
            for kk in range(0, _SLAB, 4):
                j = base + _SLAB * p + kk
                idxrows = [dsl.at[kk + q] for q in range(4)]
                g_wait(rows_a, gsa)
                s_start(rows_a, idxrows[0], ssa)
                g_wait(rows_b, gsb)
                s_start(rows_b, idxrows[1], ssb)
                s_wait(rows_a, idxrows[0], ssa)

                @pl.when(j + 4 < _NCH)
                def _():
                    g_start(j + 4, rows_a, gsa)

                g_wait(rows_c, gsc)
                s_start(rows_c, idxrows[2], ssc)
                s_wait(rows_b, idxrows[1], ssb)

                @pl.when(j + 5 < _NCH)
                def _():
                    g_start(j + 5, rows_b, gsb)

                g_wait(rows_d, gsd)
                s_start(rows_d, idxrows[3], ssd)
                s_wait(rows_c, idxrows[2], ssc)

                @pl.when(j + 6 < _NCH)
                def _():
                    g_start(j + 6, rows_c, gsc)

                s_wait(rows_d, idxrows[3], ssd)

                @pl.when(j + 7 < _NCH)
                def _():
                    g_start(j + 7, rows_d, gsd)

            nxt = 2 * (i + 1) + p

            @pl.when(nxt < _NSLAB)
            def _():
                pltpu.async_copy(dst_hbm.at[wid, nxt], dsl, dsem)

        return carry

    lax.fori_loop(0, _NSLAB // 2, body, 0)
    plsc.subcore_barrier()
    pltpu.sync_copy(acc_sh.at[pl.ds(sid * _ROWS_W, _ROWS_W)],
                    out_hbm.at[cid, pl.ds(sid * _ROWS_W, _ROWS_W)])


def _tc1_body(x_ref, w_ref, d0_ref, d1_ref, g_ref, dinv_ref):
    deg = d0_ref[...] + d1_ref[...] + 1.0
    dinv = lax.rsqrt(deg)
    h = jnp.dot(x_ref[...], w_ref[...], preferred_element_type=jnp.float32)
    g_ref[...] = h * dinv
    dinv_ref[...] = dinv


def _tc2_body(s_ref, g_ref, dinv_ref, b_ref, w_ref, gn_ref):
    dinv = dinv_ref[...]
    agg = dinv * (s_ref[0] + s_ref[1] + g_ref[...]) + b_ref[...]
    h = jnp.maximum(agg, 0.0)
    hw = jnp.dot(h, w_ref[...], preferred_element_type=jnp.float32)
    gn_ref[...] = hw * dinv


def _tc3_body(s_ref, g_ref, dinv_ref, b_ref, proto_ref, wlt_ref,
              logits_ref, probs_ref, emb_ref, dist_ref):
    agg = dinv_ref[...] * (s_ref[0] + s_ref[1] + g_ref[...]) + b_ref[...]
    emb = jnp.maximum(agg, 0.0)
    proto = proto_ref[...]
    xp = lax.dot_general(emb, proto, (((1,), (1,)), ((), ())),
                         preferred_element_type=jnp.float32)
    psq = jnp.sum(proto * proto, axis=1)[None, :]
    esq = jnp.sum(emb * emb, axis=1, keepdims=True)
    dist = -2.0 * xp + esq + psq
    sim = jnp.log((dist + 1.0) / (dist + 1e-4))
    logits = jnp.dot(sim, wlt_ref[...], preferred_element_type=jnp.float32)
    col = lax.broadcasted_iota(jnp.int32, logits.shape, 1)
    ml = jnp.where(col < 10, logits, -1e30)
    m = jnp.max(ml, axis=1, keepdims=True)
    e = jnp.exp(ml - m)
    p = e / jnp.sum(e, axis=1, keepdims=True)
    logits_ref[...] = logits[:, :10]
    probs_ref[...] = p[:, :10]
    emb_ref[...] = emb
    dist_ref[...] = dist[:, :50]


_BLK = 1000
_GRID = _N // _BLK

_row_spec = pl.BlockSpec((_BLK, _D), lambda i: (i, 0))
_col_spec = pl.BlockSpec((_BLK, 1), lambda i: (i, 0))
_w_spec = pl.BlockSpec((_D, _D), lambda i: (0, 0))
_b_spec = pl.BlockSpec((1, _D), lambda i: (0, 0))
_s_spec = pl.BlockSpec((_NC, _BLK, _D), lambda i: (0, i, 0))

_tc1 = pl.pallas_call(
    _tc1_body,
    grid=(_GRID,),
    in_specs=[_row_spec, _w_spec, _col_spec, _col_spec],
    out_specs=[_row_spec, _col_spec],
    out_shape=[jax.ShapeDtypeStruct((_N, _D), jnp.float32),
               jax.ShapeDtypeStruct((_N, 1), jnp.float32)],
)

_tc2 = pl.pallas_call(
    _tc2_body,
    grid=(_GRID,),
    in_specs=[_s_spec, _row_spec, _col_spec, _b_spec, _w_spec],
    out_specs=[_row_spec],
    out_shape=[jax.ShapeDtypeStruct((_N, _D), jnp.float32)],
)

_tc3 = pl.pallas_call(
    _tc3_body,
    grid=(_GRID,),
    in_specs=[_s_spec, _row_spec, _col_spec, _b_spec,
              pl.BlockSpec((64, _D), lambda i: (0, 0)),
              pl.BlockSpec((64, 16), lambda i: (0, 0))],
    out_specs=[pl.BlockSpec((_BLK, 10), lambda i: (i, 0)),
               pl.BlockSpec((_BLK, 10), lambda i: (i, 0)),
               _row_spec,
               pl.BlockSpec((_BLK, 50), lambda i: (i, 0))],
    out_shape=[jax.ShapeDtypeStruct((_N, 10), jnp.float32),
               jax.ShapeDtypeStruct((_N, 10), jnp.float32),
               jax.ShapeDtypeStruct((_N, _D), jnp.float32),
               jax.ShapeDtypeStruct((_N, 50), jnp.float32)],
)


def kernel(x, edge_index, W1, b1, W2, b2, W3, b3, proto, W_last):
    e0 = edge_index[0].astype(jnp.int32).reshape(_NW, _EW)
    e1 = edge_index[1].astype(jnp.int32).reshape(_NW, _EW)
    pad_src = jnp.broadcast_to((jnp.arange(_EPAD, dtype=jnp.int32) * 89) % _N,
                               (_NW, _EPAD))
    pad_dst = jnp.broadcast_to(
        _N + 16 + (jnp.arange(_EPAD, dtype=jnp.int32) % 224), (_NW, _EPAD))
    src = jnp.concatenate([e0, pad_src], axis=1).reshape(_NW, 1, _EWP)
    dst = jnp.concatenate([e1, pad_dst], axis=1).reshape(
        _NW, _NSLAB, _SLAB, _CH)

    _sc_degree, _sc_scatter = _sc_kernels()
    deg2 = _sc_degree(dst)
    deg0 = deg2[0, 0, :_N].reshape(_N, 1)
    deg1 = deg2[1, 0, :_N].reshape(_N, 1)

    b1r = b1.reshape(1, _D)
    b2r = b2.reshape(1, _D)
    b3r = b3.reshape(1, _D)
    proto_pad = jnp.zeros((64, _D), jnp.float32).at[:50].set(proto)
    wlt_pad = jnp.zeros((64, 16), jnp.float32).at[:50, :10].set(W_last.T)

    g1, dinv = _tc1(x, W1, deg0, deg1)

    s = _sc_scatter(g1, src, dst)
    g2, = _tc2(s, g1, dinv, b1r, W2)

    s = _sc_scatter(g2, src, dst)
    g3, = _tc2(s, g2, dinv, b2r, W3)

    s = _sc_scatter(g3, src, dst)
    logits, probs, emb, dist = _tc3(s, g3, dinv, b3r, proto_pad, wlt_pad)
    return (logits, probs, emb, dist)
